# baseline (speedup 1.0000x reference)
import jax
import jax.numpy as jnp
from jax.experimental import pallas as pl
from jax.experimental.pallas import tpu as pltpu

_CB = 4
_OD = 3


def _softmax_t_kernel(a_ref, att_ref):
    a = a_ref[...]
    m = jnp.max(a, axis=1, keepdims=True)
    e = jnp.exp(a - m)
    att_ref[...] = (e / jnp.sum(e, axis=1, keepdims=True)).T


def _mm_kernel(att_ref, x_ref, o_ref, obuf, outsem):
    nb = x_ref.shape[0]
    ncb = o_ref.shape[0] // _CB
    i = pl.program_id(0)
    attT = att_ref[...]
    slot = jax.lax.rem(i, _OD)

    def out_copy(j, s, cc):
        return pltpu.make_async_copy(
            obuf.at[s, cc], o_ref.at[j * _CB + cc], outsem.at[s])

    @pl.when(i >= _OD)
    def _():
        for cc in range(_CB):
            out_copy(i - _OD, slot, cc).wait()

    for cc in range(_CB):
        for nn in range(nb):
            obuf[slot, cc, nn] = jnp.dot(
                x_ref[nn, cc], attT, preferred_element_type=jnp.float32)

    for cc in range(_CB):
        out_copy(i, slot, cc).start()

    @pl.when(i == ncb - 1)
    def _():
        for k in range(_OD):
            j = ncb - _OD + k
            for cc in range(_CB):
                out_copy(j, j % _OD, cc).wait()


def kernel(x, edge_list):
    n, c, v, l = x.shape
    xT = jnp.swapaxes(x, 2, 3)

    attT = pl.pallas_call(
        _softmax_t_kernel,
        out_shape=jax.ShapeDtypeStruct((v, v), jnp.float32),
    )(edge_list)

    ncb = c // _CB
    ot = pl.pallas_call(
        _mm_kernel,
        grid=(ncb,),
        in_specs=[
            pl.BlockSpec((v, v), lambda i: (0, 0)),
            pl.BlockSpec((n, _CB, l, v), lambda i: (0, i, 0, 0)),
        ],
        out_specs=pl.BlockSpec(memory_space=pltpu.MemorySpace.HBM),
        out_shape=jax.ShapeDtypeStruct((c, n, l, v), jnp.float32),
        scratch_shapes=[
            pltpu.VMEM((_OD, _CB, n, l, v), jnp.float32),
            pltpu.SemaphoreType.DMA((_OD,)),
        ],
        compiler_params=pltpu.CompilerParams(
            dimension_semantics=("arbitrary",),
        ),
    )(attT, xT)
    return jnp.transpose(ot, (0, 1, 3, 2)).reshape(c, n * v, l)
